# baseline (speedup 1.0000x reference)
import functools

import jax
import jax.numpy as jnp
from jax import lax
from jax.experimental import pallas as pl
from jax.experimental.pallas import tpu as pltpu
from jax.experimental.pallas import tpu_sc as plsc

B, L = 4096, 50
D = 64
V = 100000
NC, NS = 2, 16
NW = NC * NS
BPW = B // NW
K = 4
CC = K * L
NCHUNK = BPW // K

_mesh = plsc.VectorSubcoreMesh(core_axis_name="c", subcore_axis_name="s")


@functools.partial(
    pl.kernel,
    mesh=_mesh,
    out_type=jax.ShapeDtypeStruct((B, L, D), jnp.float32),
    compiler_params=pltpu.CompilerParams(use_tc_tiling_on_sc=False),
    scratch_types=[
        pltpu.VMEM((BPW, L), jnp.int32),
        pltpu.VMEM((BPW, L), jnp.int32),
        pltpu.VMEM((BPW, L), jnp.int32),
        pltpu.VMEM((BPW * L,), jnp.int32),
        pltpu.VMEM((BPW * L,), jnp.int32),
        pltpu.VMEM((BPW * L,), jnp.int32),
        pltpu.VMEM((2, CC, D), jnp.float32),
        pltpu.VMEM((2, CC, D), jnp.float32),
        pltpu.VMEM((2, CC, D), jnp.float32),
        pltpu.SemaphoreType.DMA,
        pltpu.SemaphoreType.DMA,
        pltpu.SemaphoreType.DMA,
        pltpu.SemaphoreType.DMA,
        pltpu.SemaphoreType.DMA,
        pltpu.SemaphoreType.DMA,
    ],
)
def _triple_embed(oid, tid, cid, tab, out,
                  i1, i2, i3, c1, c2, c3, b1, b2, b3,
                  sa1, sa2, sa3, sb1, sb2, sb3):
    wid = lax.axis_index("s") * NC + lax.axis_index("c")
    wb = wid * BPW

    pltpu.sync_copy(oid.at[pl.ds(wb, BPW)], i1)
    pltpu.sync_copy(tid.at[pl.ds(wb, BPW)], i2)
    pltpu.sync_copy(cid.at[pl.ds(wb, BPW)], i3)

    def compact(r, cc):
        base = r * L
        for src, dst, shift in ((i1, c1, 0), (i2, c2, V), (i3, c3, 2 * V)):
            for off in (0, 16, 32, 34):
                v = src[r, pl.ds(off, 16)]
                if shift:
                    v = v + shift
                dst[pl.ds(base + off, 16)] = v
        return cc

    lax.fori_loop(0, BPW, compact, 0)

    sems = ((sa1, sa2, sa3), (sb1, sb2, sb3))

    def fire(c, k):
        s1, s2, s3 = sems[k]
        isl = pl.ds(c * CC, CC)
        pltpu.async_copy(tab.at[c1.at[isl]], b1.at[k], s1)
        pltpu.async_copy(tab.at[c2.at[isl]], b2.at[k], s2)
        pltpu.async_copy(tab.at[c3.at[isl]], b3.at[k], s3)

    def drain(c, k):
        s1, s2, s3 = sems[k]
        isl = pl.ds(c * CC, CC)
        pltpu.make_async_copy(tab.at[c1.at[isl]], b1.at[k], s1).wait()
        pltpu.make_async_copy(tab.at[c2.at[isl]], b2.at[k], s2).wait()
        pltpu.make_async_copy(tab.at[c3.at[isl]], b3.at[k], s3).wait()

        def row(r, cc):
            for j in range(D // 16):
                sl = pl.ds(j * 16, 16)
                b1[k, r, sl] = b1[k, r, sl] + b2[k, r, sl] + b3[k, r, sl]
            return cc

        lax.fori_loop(0, CC, row, 0)
        for j in range(K):
            pltpu.sync_copy(b1.at[k, pl.ds(j * L, L)], out.at[wb + c * K + j])

    fire(0, 0)

    def body(h, carry):
        ca = 2 * h
        fire(ca + 1, 1)
        drain(ca, 0)

        @pl.when(ca + 2 < NCHUNK)
        def _():
            fire(ca + 2, 0)

        drain(ca + 1, 1)
        return carry

    lax.fori_loop(0, NCHUNK // 2, body, 0)


def kernel(out_ids, tree_ids, ctx_ids, out_table, tree_table, ctx_table):
    one = (1 + 0 * out_ids[0, 0]).astype(jnp.float32)
    tab = jnp.concatenate([out_table, tree_table, ctx_table], axis=0) * one
    res = _triple_embed(out_ids.astype(jnp.int32), tree_ids.astype(jnp.int32),
                        ctx_ids.astype(jnp.int32), tab)
    return res * one


# speedup vs baseline: 1.7154x; 1.7154x over previous
import functools

import jax
import jax.numpy as jnp
from jax import lax
from jax.experimental import pallas as pl
from jax.experimental.pallas import tpu as pltpu
from jax.experimental.pallas import tpu_sc as plsc

B, L = 4096, 50
D = 64
LPAD, DPAD = 56, 128
N = B * L
NC, NS = 2, 16
NW = NC * NS
RPW = N // NW
K = 4
CC = K * L
NCHUNK = RPW // CC

_mesh = plsc.VectorSubcoreMesh(core_axis_name="c", subcore_axis_name="s")


@functools.partial(
    pl.kernel,
    mesh=_mesh,
    out_type=jax.ShapeDtypeStruct((B, LPAD, DPAD), jnp.float32),
    compiler_params=pltpu.CompilerParams(use_tc_tiling_on_sc=False),
    scratch_types=[
        pltpu.VMEM((RPW,), jnp.int32),
        pltpu.VMEM((RPW,), jnp.int32),
        pltpu.VMEM((RPW,), jnp.int32),
        pltpu.VMEM((2, CC, D), jnp.float32),
        pltpu.VMEM((2, CC, D), jnp.float32),
        pltpu.VMEM((2, CC, D), jnp.float32),
        pltpu.SemaphoreType.DMA,
        pltpu.SemaphoreType.DMA,
        pltpu.SemaphoreType.DMA,
        pltpu.SemaphoreType.DMA,
        pltpu.SemaphoreType.DMA,
        pltpu.SemaphoreType.DMA,
    ],
)
def _triple_embed(oid, tid, cid, t1, t2, t3, out,
                  i1, i2, i3, b1, b2, b3, sa1, sa2, sa3, sb1, sb2, sb3):
    wid = lax.axis_index("s") * NC + lax.axis_index("c")
    wb = wid * RPW

    pltpu.sync_copy(oid.at[pl.ds(wb, RPW)], i1)
    pltpu.sync_copy(tid.at[pl.ds(wb, RPW)], i2)
    pltpu.sync_copy(cid.at[pl.ds(wb, RPW)], i3)

    sems = ((sa1, sa2, sa3), (sb1, sb2, sb3))

    def fire(c, k):
        s1, s2, s3 = sems[k]
        isl = pl.ds(c * CC, CC)
        pltpu.async_copy(t1.at[i1.at[isl]], b1.at[k], s1)
        pltpu.async_copy(t2.at[i2.at[isl]], b2.at[k], s2)
        pltpu.async_copy(t3.at[i3.at[isl]], b3.at[k], s3)

    def drain(c, k):
        s1, s2, s3 = sems[k]
        isl = pl.ds(c * CC, CC)
        pltpu.make_async_copy(t1.at[i1.at[isl]], b1.at[k], s1).wait()
        pltpu.make_async_copy(t2.at[i2.at[isl]], b2.at[k], s2).wait()
        pltpu.make_async_copy(t3.at[i3.at[isl]], b3.at[k], s3).wait()

        def row(r, cc):
            for j in range(D // 16):
                sl = pl.ds(j * 16, 16)
                b1[k, r, sl] = b1[k, r, sl] + b2[k, r, sl] + b3[k, r, sl]
            return cc

        lax.fori_loop(0, CC, row, 0)
        bb = wid * (B // NW) + c * K
        for j in range(K):
            pltpu.sync_copy(b1.at[k, pl.ds(j * L, L)],
                            out.at[bb + j, pl.ds(0, L), pl.ds(0, D)])

    fire(0, 0)

    def body(h, carry):
        ca = 2 * h
        fire(ca + 1, 1)
        drain(ca, 0)

        @pl.when(ca + 2 < NCHUNK)
        def _():
            fire(ca + 2, 0)

        drain(ca + 1, 1)
        return carry

    lax.fori_loop(0, NCHUNK // 2, body, 0)


def kernel(out_ids, tree_ids, ctx_ids, out_table, tree_table, ctx_table):
    oid = out_ids.reshape(-1).astype(jnp.int32)
    tid = tree_ids.reshape(-1).astype(jnp.int32)
    cid = ctx_ids.reshape(-1).astype(jnp.int32)
    res = _triple_embed(oid, tid, cid, out_table, tree_table, ctx_table)
    return lax.slice(res, (0, 0, 0), (B, L, D))


# speedup vs baseline: 1.7339x; 1.0108x over previous
import functools

import jax
import jax.numpy as jnp
from jax import lax
from jax.experimental import pallas as pl
from jax.experimental.pallas import tpu as pltpu
from jax.experimental.pallas import tpu_sc as plsc

B, L = 4096, 50
D = 64
LPAD, DPAD = 56, 128
N = B * L
NC, NS = 2, 16
NW = NC * NS
RPW = N // NW
K = 4
CC = K * L
NCHUNK = RPW // CC
P = 4

_mesh = plsc.VectorSubcoreMesh(core_axis_name="c", subcore_axis_name="s")


@functools.partial(
    pl.kernel,
    mesh=_mesh,
    out_type=jax.ShapeDtypeStruct((B, LPAD, DPAD), jnp.float32),
    compiler_params=pltpu.CompilerParams(use_tc_tiling_on_sc=False),
    scratch_types=[
        pltpu.VMEM((RPW,), jnp.int32),
        pltpu.VMEM((RPW,), jnp.int32),
        pltpu.VMEM((RPW,), jnp.int32),
        pltpu.VMEM((P, CC, D), jnp.float32),
        pltpu.SemaphoreType.DMA,
        pltpu.SemaphoreType.DMA,
        pltpu.SemaphoreType.DMA,
        pltpu.SemaphoreType.DMA,
        pltpu.SemaphoreType.DMA,
        pltpu.SemaphoreType.DMA,
        pltpu.SemaphoreType.DMA,
        pltpu.SemaphoreType.DMA,
        pltpu.SemaphoreType.DMA,
        pltpu.SemaphoreType.DMA,
        pltpu.SemaphoreType.DMA,
        pltpu.SemaphoreType.DMA,
    ],
)
def _triple_embed(oid, tid, cid, t1, t2, t3, out,
                  i1, i2, i3, acc,
                  sa0, sa1, sa2, sa3, sb0, sb1, sb2, sb3,
                  sd0, sd1, sd2, sd3):
    wid = lax.axis_index("s") * NC + lax.axis_index("c")
    wb = wid * RPW

    pltpu.sync_copy(oid.at[pl.ds(wb, RPW)], i1)
    pltpu.sync_copy(tid.at[pl.ds(wb, RPW)], i2)
    pltpu.sync_copy(cid.at[pl.ds(wb, RPW)], i3)

    sa = (sa0, sa1, sa2, sa3)
    sb = (sb0, sb1, sb2, sb3)
    sd = (sd0, sd1, sd2, sd3)

    def f1(c, p):
        pltpu.async_copy(t1.at[i1.at[pl.ds(c * CC, CC)]], acc.at[p], sa[p])

    def f23(c, p):
        isl = pl.ds(c * CC, CC)
        pltpu.make_async_copy(t1.at[i1.at[isl]], acc.at[p], sa[p]).wait()
        pltpu.async_copy(t2.at[i2.at[isl]], acc.at[p], sb[p], add=True)
        pltpu.async_copy(t3.at[i3.at[isl]], acc.at[p], sb[p], add=True)

    def wbf(c, p):
        isl = pl.ds(c * CC, CC)
        pltpu.make_async_copy(t2.at[i2.at[isl]], acc.at[p], sb[p]).wait()
        pltpu.make_async_copy(t3.at[i3.at[isl]], acc.at[p], sb[p]).wait()
        bb = wid * (B // NW) + c * K
        for j in range(K):
            pltpu.async_copy(acc.at[p, pl.ds(j * L, L)],
                             out.at[bb + j, pl.ds(0, L), pl.ds(0, D)], sd[p])

    def wbw(c, p):
        bb = wid * (B // NW) + c * K
        for j in range(K):
            pltpu.make_async_copy(acc.at[p, pl.ds(j * L, L)],
                                  out.at[bb + j, pl.ds(0, L), pl.ds(0, D)],
                                  sd[p]).wait()

    f1(0, 0)
    f1(1, 1)
    f1(2, 2)
    f23(0, 0)
    f1(3, 3)
    f23(1, 1)
    wbf(0, 0)

    def body(h, carry):
        s0 = 4 * h
        for q in range(4):
            s = s0 + q
            wbw(s - 4, q)
            f1(s, q)
            f23(s - 2, (q + 2) % 4)
            wbf(s - 3, (q + 1) % 4)
        return carry

    lax.fori_loop(1, NCHUNK // 4, body, 0)

    wbw(NCHUNK - 4, 0)
    f23(NCHUNK - 2, 2)
    wbf(NCHUNK - 3, 1)
    wbw(NCHUNK - 3, 1)
    f23(NCHUNK - 1, 3)
    wbf(NCHUNK - 2, 2)
    wbw(NCHUNK - 2, 2)
    wbf(NCHUNK - 1, 3)
    wbw(NCHUNK - 1, 3)


def kernel(out_ids, tree_ids, ctx_ids, out_table, tree_table, ctx_table):
    oid = out_ids.reshape(-1).astype(jnp.int32)
    tid = tree_ids.reshape(-1).astype(jnp.int32)
    cid = ctx_ids.reshape(-1).astype(jnp.int32)
    res = _triple_embed(oid, tid, cid, out_table, tree_table, ctx_table)
    return lax.slice(res, (0, 0, 0), (B, L, D))
